# baseline (speedup 1.0000x reference)
import functools
import math

import jax
import jax.numpy as jnp
import numpy as np
from jax import lax
from jax.experimental import pallas as pl
from jax.experimental.pallas import tpu as pltpu
from jax.experimental.pallas import tpu_sc as plsc

NC = 2
NS = 16
NW = NC * NS
LANES = 16
CHUNK = 128


def _round_up(x, m):
    return ((x + m - 1) // m) * m


def _t1_body(x_ref, nt_ref, lutW_ref, lutb_ref, Wk_ref, bk_ref, Wq_ref,
             bq_ref, Wv_ref, bv_ref, KBD_ref, VBD_ref,
             q_out, krot_out, vrot_out, h_out):
    T = lutW_ref.shape[0]
    R = KBD_ref.shape[0]
    xb = x_ref[...]
    nt = nt_ref[...]
    masks = [(1 - jnp.minimum(jnp.abs(nt - t), 1)).astype(jnp.float32)
             for t in range(T)]
    h = xb * masks[0]
    for t in range(1, T):
        m = jnp.dot(xb, lutW_ref[t], preferred_element_type=jnp.float32)
        m = m + lutb_ref[t]
        h = h + m * masks[t]
    h_out[...] = h

    def sel_proj(W_ref, b_ref):
        out = None
        for t in range(T):
            m = jnp.dot(h, W_ref[t], preferred_element_type=jnp.float32)
            m = (m + b_ref[t]) * masks[t]
            out = m if out is None else out + m
        return out

    k = sel_proj(Wk_ref, bk_ref)
    q = sel_proj(Wq_ref, bq_ref)
    v = sel_proj(Wv_ref, bv_ref)
    q_out[...] = q
    for r in range(R):
        krot_out[r] = jnp.dot(k, KBD_ref[r], preferred_element_type=jnp.float32)
        vrot_out[r] = jnp.dot(v, VBD_ref[r], preferred_element_type=jnp.float32)


def _t1_call(x, nt2, lutW, lutb, Wk, bk, Wq, bq, Wv, bv, KBD, VBD):
    N, DIN = x.shape
    R = KBD.shape[0]
    F = KBD.shape[2]
    B = 1000 if N % 1000 == 0 else N
    grid = (N // B,)
    full = lambda a: pl.BlockSpec(a.shape, lambda i: (0,) * a.ndim)
    return pl.pallas_call(
        _t1_body,
        grid=grid,
        in_specs=[
            pl.BlockSpec((B, DIN), lambda i: (i, 0)),
            pl.BlockSpec((B, 1), lambda i: (i, 0)),
            full(lutW), full(lutb), full(Wk), full(bk), full(Wq), full(bq),
            full(Wv), full(bv), full(KBD), full(VBD),
        ],
        out_specs=[
            pl.BlockSpec((B, F), lambda i: (i, 0)),
            pl.BlockSpec((R, B, F), lambda i: (0, i, 0)),
            pl.BlockSpec((R, B, F), lambda i: (0, i, 0)),
            pl.BlockSpec((B, DIN), lambda i: (i, 0)),
        ],
        out_shape=[
            jax.ShapeDtypeStruct((N, F), jnp.float32),
            jax.ShapeDtypeStruct((R, N, F), jnp.float32),
            jax.ShapeDtypeStruct((R, N, F), jnp.float32),
            jax.ShapeDtypeStruct((N, DIN), jnp.float32),
        ],
    )(x, nt2, lutW, lutb, Wk, bk, Wq, bq, Wv, bv, KBD, VBD)


VW = 128


def _make_sc_edge(N, NPAD, E, F, H, CHUNK=CHUNK):
    Dh = F // H
    assert Dh % LANES == 0 and F % LANES == 0
    split_den = F + LANES > VW
    assert E % CHUNK == 0
    nch = E // CHUNK
    base_ch, extra_ch = nch // NW, nch % NW
    zrows = NPAD // NS
    assert NPAD % (NS * 8) == 0
    NP8 = NPAD // 8
    wrows = NP8 // NS
    mesh = plsc.VectorSubcoreMesh(
        core_axis_name="c", subcore_axis_name="s", num_cores=NC,
        num_subcores=NS)

    out_type = [jax.ShapeDtypeStruct((NC, NPAD, VW), jnp.float32)]
    scratch = [
        pltpu.VMEM((CHUNK,), jnp.int32),
        pltpu.VMEM((CHUNK,), jnp.int32),
        pltpu.VMEM((CHUNK,), jnp.int32),
        pltpu.VMEM((CHUNK,), jnp.int32),
        pltpu.VMEM((CHUNK, F), jnp.float32),
        pltpu.VMEM((CHUNK, F), jnp.float32),
    ] + ([] if split_den else [
        pltpu.VMEM((CHUNK, F), jnp.float32),
    ]) + [
        pltpu.VMEM((CHUNK, VW), jnp.float32),
        pltpu.VMEM_SHARED((NPAD, VW), jnp.float32),
        pltpu.SemaphoreType.DMA,
        pltpu.SemaphoreType.DMA,
        pltpu.SemaphoreType.DMA,
    ]
    if split_den:
        assert F == VW and NP8 % (NS * 8) == 0
        out_type.append(jax.ShapeDtypeStruct((NC, NP8, VW), jnp.float32))
        scratch += [
            pltpu.VMEM((CHUNK,), jnp.int32),
            pltpu.VMEM((CHUNK, VW), jnp.float32),
            pltpu.VMEM_SHARED((NP8, VW), jnp.float32),
        ]
    else:
        assert F + LANES <= VW

    @functools.partial(pl.kernel, out_type=out_type, mesh=mesh,
                       scratch_types=scratch)
    def edge_kernel(*refs):
        krot_hbm, vrot_hbm, q_hbm, src_hbm, dst_hbm, et_hbm, zeros_hbm = \
            refs[:7]
        if split_den:
            out_hbm, outw_hbm = refs[7:9]
            (srcv, dstv, etv, gidxv, krv, qdv, rowsv, acc_sh,
             sem1, sem2, sem3, dst8v, denrows, accw_sh) = refs[9:]
            vrv = rowsv
        else:
            out_hbm = refs[7]
            (srcv, dstv, etv, gidxv, krv, qdv, vrv, rowsv, acc_sh,
             sem1, sem2, sem3) = refs[8:]

        cid = lax.axis_index("c")
        sid = lax.axis_index("s")
        wid = sid * NC + cid
        pltpu.sync_copy(zeros_hbm.at[pl.ds(0, zrows)],
                        acc_sh.at[pl.ds(sid * zrows, zrows)])
        if split_den:
            pltpu.sync_copy(zeros_hbm.at[pl.ds(0, wrows)],
                            accw_sh.at[pl.ds(sid * wrows, wrows)])
            pltpu.sync_copy(zeros_hbm.at[pl.ds(0, CHUNK)], denrows)
        else:
            pltpu.sync_copy(zeros_hbm.at[pl.ds(0, CHUNK)], rowsv)
        plsc.subcore_barrier()

        iota = lax.iota(jnp.int32, LANES)
        perms = [iota ^ k for k in (8, 4, 2, 1)]
        onehots = [(1 - jnp.minimum(jnp.abs(iota - h), 1)).astype(jnp.float32)
                   for h in range(H)]

        def hsum(v):
            for p in perms:
                v = v + v.at[p].get(mode='promise_in_bounds')
            return v

        nchunks = base_ch + jnp.where(wid < extra_ch, 1, 0)

        def chunk_body(j, carry):
            base = (wid + j * NW) * CHUNK
            base = pl.multiple_of(base, 8)
            pltpu.sync_copy(src_hbm.at[pl.ds(base, CHUNK)], srcv)
            pltpu.sync_copy(dst_hbm.at[pl.ds(base, CHUNK)], dstv)
            pltpu.sync_copy(et_hbm.at[pl.ds(base, CHUNK)], etv)
            for i in range(CHUNK // LANES):
                sl = pl.ds(i * LANES, LANES)
                gidxv[sl] = etv[sl] * N + srcv[sl]
                if split_den:
                    dst8v[sl] = dstv[sl] >> 3
            cp1 = pltpu.async_copy(krot_hbm.at[gidxv], krv, sem1)
            cp2 = pltpu.async_copy(q_hbm.at[dstv], qdv, sem2)
            cp3 = pltpu.async_copy(vrot_hbm.at[gidxv], vrv, sem3)
            cp1.wait()
            cp2.wait()
            cp3.wait()

            def edge_body(e, c2):
                whs = []
                for h in range(H):
                    acc = None
                    for c in range(h * Dh // LANES, (h + 1) * Dh // LANES):
                        p = (krv[e, pl.ds(c * LANES, LANES)] *
                             qdv[e, pl.ds(c * LANES, LANES)])
                        acc = p if acc is None else acc + p
                    whs.append(jnp.exp(hsum(acc)))
                for c in range(F // LANES):
                    vv = vrv[e, pl.ds(c * LANES, LANES)]
                    rowsv[e, pl.ds(c * LANES, LANES)] = (
                        vv * whs[(c * LANES) // Dh])
                wcol = None
                for h in range(H):
                    term = whs[h] * onehots[h]
                    wcol = term if wcol is None else wcol + term
                if split_den:
                    slot = (dstv[pl.ds(e, 1)][0] & 7) * LANES
                    denrows[e, pl.ds(slot, LANES)] = wcol
                else:
                    rowsv[e, pl.ds(F, LANES)] = wcol
                return c2

            lax.fori_loop(0, CHUNK, edge_body, 0)
            pltpu.sync_copy(rowsv, acc_sh.at[dstv], add=True)
            if split_den:
                pltpu.sync_copy(denrows, accw_sh.at[dst8v], add=True)

                zcol = onehots[0] * 0.0

                def clr_body(e, c2):
                    slot = (dstv[pl.ds(e, 1)][0] & 7) * LANES
                    denrows[e, pl.ds(slot, LANES)] = zcol
                    return c2

                lax.fori_loop(0, CHUNK, clr_body, 0)
            return carry

        lax.fori_loop(0, nchunks, chunk_body, 0)
        plsc.subcore_barrier()
        pltpu.sync_copy(acc_sh.at[pl.ds(sid * zrows, zrows)],
                        out_hbm.at[cid].at[pl.ds(sid * zrows, zrows)])
        if split_den:
            pltpu.sync_copy(accw_sh.at[pl.ds(sid * wrows, wrows)],
                            outw_hbm.at[cid].at[pl.ds(sid * wrows, wrows)])

    return edge_kernel


def _make_sc_edge_packed(N, NPAD, E, F, H, CHUNK=CHUNK):
    Dh = F // H
    assert Dh % LANES == 0 and 2 * F + LANES <= VW
    assert E % CHUNK == 0
    nch = E // CHUNK
    base_ch, extra_ch = nch // NW, nch % NW
    zrows = NPAD // NS
    mesh = plsc.VectorSubcoreMesh(
        core_axis_name="c", subcore_axis_name="s", num_cores=NC,
        num_subcores=NS)

    out_type = [jax.ShapeDtypeStruct((NC, NPAD, VW), jnp.float32)]
    scratch = [
        pltpu.VMEM((CHUNK,), jnp.int32),
        pltpu.VMEM((CHUNK,), jnp.int32),
        pltpu.VMEM((CHUNK,), jnp.int32),
        pltpu.VMEM((CHUNK,), jnp.int32),
        pltpu.VMEM((CHUNK, VW), jnp.float32),
        pltpu.VMEM((CHUNK, VW), jnp.float32),
        pltpu.VMEM((CHUNK, VW), jnp.float32),
        pltpu.VMEM_SHARED((NPAD, VW), jnp.float32),
        pltpu.SemaphoreType.DMA,
        pltpu.SemaphoreType.DMA,
    ]

    @functools.partial(pl.kernel, out_type=out_type, mesh=mesh,
                       scratch_types=scratch)
    def edge_kernel(kv_hbm, q_hbm, src_hbm, dst_hbm, et_hbm, zeros_hbm,
                    out_hbm, srcv, dstv, etv, gidxv, kvv, qdv, rowsv, acc_sh,
                    sem1, sem2):
        cid = lax.axis_index("c")
        sid = lax.axis_index("s")
        wid = sid * NC + cid
        pltpu.sync_copy(zeros_hbm.at[pl.ds(0, zrows)],
                        acc_sh.at[pl.ds(sid * zrows, zrows)])
        pltpu.sync_copy(zeros_hbm.at[pl.ds(0, CHUNK)], rowsv)
        plsc.subcore_barrier()

        iota = lax.iota(jnp.int32, LANES)
        perms = [iota ^ k for k in (8, 4, 2, 1)]
        onehots = [(1 - jnp.minimum(jnp.abs(iota - h), 1)).astype(jnp.float32)
                   for h in range(H)]

        def hsum(v):
            for p in perms:
                v = v + v.at[p].get(mode='promise_in_bounds')
            return v

        nchunks = base_ch + jnp.where(wid < extra_ch, 1, 0)

        def chunk_body(j, carry):
            base = (wid + j * NW) * CHUNK
            base = pl.multiple_of(base, 8)
            pltpu.sync_copy(src_hbm.at[pl.ds(base, CHUNK)], srcv)
            pltpu.sync_copy(dst_hbm.at[pl.ds(base, CHUNK)], dstv)
            pltpu.sync_copy(et_hbm.at[pl.ds(base, CHUNK)], etv)
            for i in range(CHUNK // LANES):
                sl = pl.ds(i * LANES, LANES)
                gidxv[sl] = etv[sl] * N + srcv[sl]
            cp1 = pltpu.async_copy(kv_hbm.at[gidxv], kvv, sem1)
            cp2 = pltpu.async_copy(q_hbm.at[dstv], qdv, sem2)
            cp1.wait()
            cp2.wait()

            def edge_body(e, c2):
                whs = []
                for h in range(H):
                    acc = None
                    for c in range(h * Dh // LANES, (h + 1) * Dh // LANES):
                        p = (kvv[e, pl.ds(c * LANES, LANES)] *
                             qdv[e, pl.ds(c * LANES, LANES)])
                        acc = p if acc is None else acc + p
                    whs.append(jnp.exp(hsum(acc)))
                for c in range(F // LANES):
                    vv = kvv[e, pl.ds(F + c * LANES, LANES)]
                    rowsv[e, pl.ds(c * LANES, LANES)] = (
                        vv * whs[(c * LANES) // Dh])
                wcol = None
                for h in range(H):
                    term = whs[h] * onehots[h]
                    wcol = term if wcol is None else wcol + term
                rowsv[e, pl.ds(F, LANES)] = wcol
                return c2

            lax.fori_loop(0, CHUNK, edge_body, 0)
            pltpu.sync_copy(rowsv, acc_sh.at[dstv], add=True)
            return carry

        lax.fori_loop(0, nchunks, chunk_body, 0)
        plsc.subcore_barrier()
        pltpu.sync_copy(acc_sh.at[pl.ds(sid * zrows, zrows)],
                        out_hbm.at[cid].at[pl.ds(sid * zrows, zrows)])

    return edge_kernel


def _t2_body(parts_ref, wn_ref, h0_ref, nt_ref, EXP_ref, Wa_ref, ba_ref,
             skip_ref, Wk_ref, bk_ref, Wq_ref, bq_ref, Wv_ref, bv_ref,
             K1_ref, V1_ref, q1_out, kv1_out):
    T = Wa_ref.shape[0]
    R = K1_ref.shape[0]
    F = EXP_ref.shape[1]
    p = parts_ref[0] + parts_ref[1]
    w = wn_ref[0] + wn_ref[1]
    den = jnp.dot(w, EXP_ref[...], preferred_element_type=jnp.float32) + 1e-9
    agg = p[:, :F] / den
    nt = nt_ref[...]
    masks = [(1 - jnp.minimum(jnp.abs(nt - t), 1)).astype(jnp.float32)
             for t in range(T)]

    def sel_proj(hin, W_ref, b_ref):
        out = None
        for t in range(T):
            m = jnp.dot(hin, W_ref[t], preferred_element_type=jnp.float32)
            m = (m + b_ref[t]) * masks[t]
            out = m if out is None else out + m
        return out

    out0 = sel_proj(agg, Wa_ref, ba_ref)
    sig = jax.nn.sigmoid(skip_ref[...])
    a = None
    for t in range(T):
        at = masks[t] * sig[t]
        a = at if a is None else a + at
    h1 = out0 * a + h0_ref[...] * (1.0 - a)

    k1 = sel_proj(h1, Wk_ref, bk_ref)
    q1 = sel_proj(h1, Wq_ref, bq_ref)
    v1 = sel_proj(h1, Wv_ref, bv_ref)
    B = q1.shape[0]
    F1 = K1_ref.shape[2]
    q1_out[...] = jnp.concatenate(
        [q1, jnp.zeros((B, VW - F1), jnp.float32)], axis=-1)
    zkv = jnp.zeros((B, VW - 2 * F1), jnp.float32)
    for r in range(R):
        kr = jnp.dot(k1, K1_ref[r], preferred_element_type=jnp.float32)
        vr = jnp.dot(v1, V1_ref[r], preferred_element_type=jnp.float32)
        kv1_out[r] = jnp.concatenate([kr, vr, zkv], axis=-1)


def _t2_call(parts, wn, h0, nt2, EXP, Wa, ba, skip2, Wk1, bk1, Wq1, bq1,
             Wv1, bv1, K1, V1):
    N = h0.shape[0]
    DIN = h0.shape[1]
    R = K1.shape[0]
    F1 = K1.shape[2]
    B = 1000 if N % 1000 == 0 else N
    grid = (N // B,)
    full = lambda a: pl.BlockSpec(a.shape, lambda i: (0,) * a.ndim)
    return pl.pallas_call(
        _t2_body,
        grid=grid,
        in_specs=[
            pl.BlockSpec((2, B, VW), lambda i: (0, i, 0)),
            pl.BlockSpec((2, B, LANES), lambda i: (0, i, 0)),
            pl.BlockSpec((B, DIN), lambda i: (i, 0)),
            pl.BlockSpec((B, 1), lambda i: (i, 0)),
            full(EXP), full(Wa), full(ba), full(skip2),
            full(Wk1), full(bk1), full(Wq1), full(bq1), full(Wv1), full(bv1),
            full(K1), full(V1),
        ],
        out_specs=[
            pl.BlockSpec((B, VW), lambda i: (i, 0)),
            pl.BlockSpec((R, B, VW), lambda i: (0, i, 0)),
        ],
        out_shape=[
            jax.ShapeDtypeStruct((N, VW), jnp.float32),
            jax.ShapeDtypeStruct((R, N, VW), jnp.float32),
        ],
    )(parts, wn, h0, nt2, EXP, Wa, ba, skip2, Wk1, bk1, Wq1, bq1, Wv1, bv1,
      K1, V1)


def _t3_body(parts_ref, nt_ref, SEL_ref, Wa_ref, ba_ref, out_ref):
    T = Wa_ref.shape[0]
    F = SEL_ref.shape[1]
    p = parts_ref[0] + parts_ref[1]
    den = jnp.dot(p, SEL_ref[...], preferred_element_type=jnp.float32) + 1e-9
    agg = p[:, :F] / den
    nt = nt_ref[...]
    out = None
    for t in range(T):
        mt = (1 - jnp.minimum(jnp.abs(nt - t), 1)).astype(jnp.float32)
        m = jnp.dot(agg, Wa_ref[t], preferred_element_type=jnp.float32)
        m = (m + ba_ref[t]) * mt
        out = m if out is None else out + m
    out_ref[...] = out


def _t3_call(parts, nt2, SEL, Wa, ba):
    N = nt2.shape[0]
    OUTF = Wa.shape[2]
    B = 1000 if N % 1000 == 0 else N
    grid = (N // B,)
    full = lambda a: pl.BlockSpec(a.shape, lambda i: (0,) * a.ndim)
    return pl.pallas_call(
        _t3_body,
        grid=grid,
        in_specs=[
            pl.BlockSpec((2, B, VW), lambda i: (0, i, 0)),
            pl.BlockSpec((B, 1), lambda i: (i, 0)),
            full(SEL), full(Wa), full(ba),
        ],
        out_specs=pl.BlockSpec((B, OUTF), lambda i: (i, 0)),
        out_shape=jax.ShapeDtypeStruct((N, OUTF), jnp.float32),
    )(parts, nt2, SEL, Wa, ba)


def _block_diag(W):
    R, H, D, _ = W.shape
    eye = jnp.eye(H, dtype=W.dtype)
    bd = W[:, :, :, None, :] * eye[None, :, None, :, None]
    return bd.reshape(R, H * D, H * D)


def _den_selector(F, H, rowW):
    Dh = F // H
    col_head = jnp.arange(F, dtype=jnp.int32) // Dh
    rows = jnp.arange(rowW, dtype=jnp.int32)
    sel = (rows[:, None] == (F + col_head[None, :])).astype(jnp.float32)
    return sel


def _den_expander(F, H):
    Dh = F // H
    col_head = jnp.arange(F, dtype=jnp.int32) // Dh
    rows = jnp.arange(LANES, dtype=jnp.int32)
    return (rows[:, None] == col_head[None, :]).astype(jnp.float32)


def kernel(x, node_type, edge_index, edge_type, lut_W, lut_b,
           Wk0, bk0, Wq0, bq0, Wv0, bv0, Watt0, Wmsg0, pri0, Wa0, ba0, skip0,
           Wk1, bk1, Wq1, bq1, Wv1, bv1, Watt1, Wmsg1, pri1, Wa1, ba1, skip1):
    N, DIN = x.shape
    E = edge_index.shape[1]
    R, H0_, D0_, _ = Watt0.shape
    _, H1_, D1_, _ = Watt1.shape
    F0 = H0_ * D0_
    F1 = H1_ * D1_

    nt2 = node_type.reshape(N, 1)
    src = edge_index[0]
    dst = edge_index[1]

    KBD0 = _block_diag(Watt0 * (pri0 / math.sqrt(D0_))[:, :, None, None])
    VBD0 = _block_diag(Wmsg0)
    K1 = _block_diag(Watt1 * (pri1 / math.sqrt(D1_))[:, :, None, None])
    V1 = _block_diag(Wmsg1)

    r3 = lambda b: b[:, None, :]
    q0, krot0, vrot0, h0 = _t1_call(
        x, nt2, lut_W, r3(lut_b), Wk0, r3(bk0), Wq0, r3(bq0), Wv0, r3(bv0),
        KBD0, VBD0)

    NPAD = _round_up(N, NS * 8 * 8)
    zeros = jnp.zeros((max(CHUNK, NPAD // NS), VW), jnp.float32)
    s1 = _make_sc_edge(N, NPAD, E, F0, H0_, CHUNK=32)
    sc1_out = s1(krot0.reshape(R * N, F0), vrot0.reshape(R * N, F0), q0,
                 src, dst, edge_type, zeros)
    if F0 + LANES > VW:
        parts1, wparts1 = sc1_out
        wn1 = wparts1.reshape(NC, NPAD, LANES)
    else:
        (parts1,) = sc1_out
        wn1 = parts1[:, :, F0:F0 + LANES]

    EXP0 = _den_expander(F0, H0_)
    q1p, kv1 = _t2_call(
        parts1[:, :N], wn1[:, :N], h0, nt2, EXP0, Wa0, r3(ba0),
        skip0[:, None], Wk1, r3(bk1), Wq1, r3(bq1), Wv1, r3(bv1), K1, V1)

    s2 = _make_sc_edge_packed(N, NPAD, E, F1, H1_, CHUNK=64)
    (parts2,) = s2(kv1.reshape(R * N, VW), q1p, src, dst, edge_type, zeros)

    SEL1 = _den_selector(F1, H1_, VW)
    out = _t3_call(parts2[:, :N], nt2, SEL1, Wa1, r3(ba1))
    return out


# speedup vs baseline: 1.4227x; 1.4227x over previous
import functools
import math

import jax
import jax.numpy as jnp
import numpy as np
from jax import lax
from jax.experimental import pallas as pl
from jax.experimental.pallas import tpu as pltpu
from jax.experimental.pallas import tpu_sc as plsc

NC = 2
NS = 16
NW = NC * NS
LANES = 16
CHUNK = 128


def _round_up(x, m):
    return ((x + m - 1) // m) * m


def _t1_body(x_ref, nt_ref, lutW_ref, lutb_ref, Wk_ref, bk_ref, Wq_ref,
             bq_ref, Wv_ref, bv_ref, KBD_ref, VBD_ref,
             q_out, krot_out, vrot_out, h_out):
    T = lutW_ref.shape[0]
    R = KBD_ref.shape[0]
    xb = x_ref[...]
    nt = nt_ref[...]
    masks = [(1 - jnp.minimum(jnp.abs(nt - t), 1)).astype(jnp.float32)
             for t in range(T)]
    h = xb * masks[0]
    for t in range(1, T):
        m = jnp.dot(xb, lutW_ref[t], preferred_element_type=jnp.float32)
        m = m + lutb_ref[t]
        h = h + m * masks[t]
    h_out[...] = h

    def sel_proj(W_ref, b_ref):
        out = None
        for t in range(T):
            m = jnp.dot(h, W_ref[t], preferred_element_type=jnp.float32)
            m = (m + b_ref[t]) * masks[t]
            out = m if out is None else out + m
        return out

    k = sel_proj(Wk_ref, bk_ref)
    q = sel_proj(Wq_ref, bq_ref)
    v = sel_proj(Wv_ref, bv_ref)
    q_out[...] = q
    for r in range(R):
        krot_out[r] = jnp.dot(k, KBD_ref[r], preferred_element_type=jnp.float32)
        vrot_out[r] = jnp.dot(v, VBD_ref[r], preferred_element_type=jnp.float32)


def _t1_call(x, nt2, lutW, lutb, Wk, bk, Wq, bq, Wv, bv, KBD, VBD):
    N, DIN = x.shape
    R = KBD.shape[0]
    F = KBD.shape[2]
    B = 1000 if N % 1000 == 0 else N
    grid = (N // B,)
    full = lambda a: pl.BlockSpec(a.shape, lambda i: (0,) * a.ndim)
    return pl.pallas_call(
        _t1_body,
        grid=grid,
        in_specs=[
            pl.BlockSpec((B, DIN), lambda i: (i, 0)),
            pl.BlockSpec((B, 1), lambda i: (i, 0)),
            full(lutW), full(lutb), full(Wk), full(bk), full(Wq), full(bq),
            full(Wv), full(bv), full(KBD), full(VBD),
        ],
        out_specs=[
            pl.BlockSpec((B, F), lambda i: (i, 0)),
            pl.BlockSpec((R, B, F), lambda i: (0, i, 0)),
            pl.BlockSpec((R, B, F), lambda i: (0, i, 0)),
            pl.BlockSpec((B, DIN), lambda i: (i, 0)),
        ],
        out_shape=[
            jax.ShapeDtypeStruct((N, F), jnp.float32),
            jax.ShapeDtypeStruct((R, N, F), jnp.float32),
            jax.ShapeDtypeStruct((R, N, F), jnp.float32),
            jax.ShapeDtypeStruct((N, DIN), jnp.float32),
        ],
    )(x, nt2, lutW, lutb, Wk, bk, Wq, bq, Wv, bv, KBD, VBD)


VW = 128


def _make_sc_edge(N, NPAD, E, F, H, CHUNK=CHUNK):
    Dh = F // H
    assert Dh % LANES == 0 and F % LANES == 0
    split_den = F + LANES > VW
    assert E % CHUNK == 0
    nch = E // CHUNK
    base_ch, extra_ch = nch // NW, nch % NW
    zrows = NPAD // NS
    assert NPAD % (NS * 8) == 0
    NP8 = NPAD // 8
    wrows = NP8 // NS
    mesh = plsc.VectorSubcoreMesh(
        core_axis_name="c", subcore_axis_name="s", num_cores=NC,
        num_subcores=NS)

    out_type = [jax.ShapeDtypeStruct((NC, NPAD, VW), jnp.float32)]
    scratch = [
        pltpu.VMEM((CHUNK,), jnp.int32),
        pltpu.VMEM((CHUNK,), jnp.int32),
        pltpu.VMEM((CHUNK,), jnp.int32),
        pltpu.VMEM((CHUNK,), jnp.int32),
        pltpu.VMEM((CHUNK, F), jnp.float32),
        pltpu.VMEM((CHUNK, F), jnp.float32),
    ] + ([] if split_den else [
        pltpu.VMEM((CHUNK, F), jnp.float32),
    ]) + [
        pltpu.VMEM((CHUNK, VW), jnp.float32),
        pltpu.VMEM_SHARED((NPAD, VW), jnp.float32),
        pltpu.SemaphoreType.DMA,
        pltpu.SemaphoreType.DMA,
        pltpu.SemaphoreType.DMA,
    ]
    if split_den:
        assert F == VW and NP8 % (NS * 8) == 0
        out_type.append(jax.ShapeDtypeStruct((NC, NP8, VW), jnp.float32))
        scratch += [
            pltpu.VMEM((CHUNK,), jnp.int32),
            pltpu.VMEM((CHUNK, LANES), jnp.float32),
            pltpu.VMEM_SHARED((NP8, VW), jnp.float32),
        ]
    else:
        assert F + LANES <= VW

    @functools.partial(pl.kernel, out_type=out_type, mesh=mesh,
                       scratch_types=scratch)
    def edge_kernel(*refs):
        krot_hbm, vrot_hbm, q_hbm, src_hbm, dst_hbm, et_hbm, zeros_hbm = \
            refs[:7]
        if split_den:
            out_hbm, outw_hbm = refs[7:9]
            (srcv, dstv, etv, gidxv, krv, qdv, rowsv, acc_sh,
             sem1, sem2, sem3, dst8v, wbuf, accw_sh) = refs[9:]
            vrv = rowsv
        else:
            out_hbm = refs[7]
            (srcv, dstv, etv, gidxv, krv, qdv, vrv, rowsv, acc_sh,
             sem1, sem2, sem3) = refs[8:]

        cid = lax.axis_index("c")
        sid = lax.axis_index("s")
        wid = sid * NC + cid
        pltpu.sync_copy(zeros_hbm.at[pl.ds(0, zrows)],
                        acc_sh.at[pl.ds(sid * zrows, zrows)])
        if split_den:
            pltpu.sync_copy(zeros_hbm.at[pl.ds(0, wrows)],
                            accw_sh.at[pl.ds(sid * wrows, wrows)])
        else:
            pltpu.sync_copy(zeros_hbm.at[pl.ds(0, CHUNK)], rowsv)
        plsc.subcore_barrier()

        iota = lax.iota(jnp.int32, LANES)
        perms = [iota ^ k for k in (8, 4, 2, 1)]
        onehots = [(1 - jnp.minimum(jnp.abs(iota - h), 1)).astype(jnp.float32)
                   for h in range(H)]

        def hsum(v):
            for p in perms:
                v = v + v.at[p].get(mode='promise_in_bounds')
            return v

        nchunks = base_ch + jnp.where(wid < extra_ch, 1, 0)

        def chunk_body(j, carry):
            base = (wid + j * NW) * CHUNK
            base = pl.multiple_of(base, 8)
            pltpu.sync_copy(src_hbm.at[pl.ds(base, CHUNK)], srcv)
            pltpu.sync_copy(dst_hbm.at[pl.ds(base, CHUNK)], dstv)
            pltpu.sync_copy(et_hbm.at[pl.ds(base, CHUNK)], etv)
            for i in range(CHUNK // LANES):
                sl = pl.ds(i * LANES, LANES)
                gidxv[sl] = etv[sl] * N + srcv[sl]
                if split_den:
                    dst8v[sl] = dstv[sl] >> 3
            cp1 = pltpu.async_copy(krot_hbm.at[gidxv], krv, sem1)
            cp2 = pltpu.async_copy(q_hbm.at[dstv], qdv, sem2)
            cp3 = pltpu.async_copy(vrot_hbm.at[gidxv], vrv, sem3)
            cp1.wait()
            cp2.wait()
            cp3.wait()

            def edge_body(e, c2):
                whs = []
                for h in range(H):
                    acc = None
                    for c in range(h * Dh // LANES, (h + 1) * Dh // LANES):
                        p = (krv[e, pl.ds(c * LANES, LANES)] *
                             qdv[e, pl.ds(c * LANES, LANES)])
                        acc = p if acc is None else acc + p
                    whs.append(jnp.exp(hsum(acc)))
                for c in range(F // LANES):
                    vv = vrv[e, pl.ds(c * LANES, LANES)]
                    rowsv[e, pl.ds(c * LANES, LANES)] = (
                        vv * whs[(c * LANES) // Dh])
                wcol = None
                for h in range(H):
                    term = whs[h] * onehots[h]
                    wcol = term if wcol is None else wcol + term
                if split_den:
                    wbuf[e, pl.ds(0, LANES)] = wcol
                else:
                    rowsv[e, pl.ds(F, LANES)] = wcol
                return c2

            lax.fori_loop(0, CHUNK, edge_body, 0)
            pltpu.sync_copy(rowsv, acc_sh.at[dstv], add=True)
            if split_den:
                def den_body(e, c2):
                    wcol = wbuf[e, pl.ds(0, LANES)]
                    ge = (e >> 4) * LANES
                    dv = dstv[pl.ds(ge, LANES)]
                    bc = dv.at[jnp.full((LANES,), e & (LANES - 1),
                                        jnp.int32)].get(
                        mode='promise_in_bounds')
                    slotv = bc & 7
                    for s in range(8):
                        m = (1 - jnp.minimum(jnp.abs(slotv - s), 1)
                             ).astype(jnp.float32)
                        rowsv[e, pl.ds(s * LANES, LANES)] = wcol * m
                    return c2

                lax.fori_loop(0, CHUNK, den_body, 0)
                pltpu.sync_copy(rowsv, accw_sh.at[dst8v], add=True)
            return carry

        lax.fori_loop(0, nchunks, chunk_body, 0)
        plsc.subcore_barrier()
        pltpu.sync_copy(acc_sh.at[pl.ds(sid * zrows, zrows)],
                        out_hbm.at[cid].at[pl.ds(sid * zrows, zrows)])
        if split_den:
            pltpu.sync_copy(accw_sh.at[pl.ds(sid * wrows, wrows)],
                            outw_hbm.at[cid].at[pl.ds(sid * wrows, wrows)])

    return edge_kernel


def _make_sc_edge_packed(N, NPAD, E, F, H, CHUNK=CHUNK):
    Dh = F // H
    assert Dh % LANES == 0 and 2 * F + LANES <= VW
    assert E % CHUNK == 0
    nch = E // CHUNK
    base_ch, extra_ch = nch // NW, nch % NW
    zrows = NPAD // NS
    mesh = plsc.VectorSubcoreMesh(
        core_axis_name="c", subcore_axis_name="s", num_cores=NC,
        num_subcores=NS)

    out_type = [jax.ShapeDtypeStruct((NC, NPAD, VW), jnp.float32)]
    scratch = [
        pltpu.VMEM((CHUNK,), jnp.int32),
        pltpu.VMEM((CHUNK,), jnp.int32),
        pltpu.VMEM((CHUNK,), jnp.int32),
        pltpu.VMEM((CHUNK,), jnp.int32),
        pltpu.VMEM((CHUNK, VW), jnp.float32),
        pltpu.VMEM((CHUNK, VW), jnp.float32),
        pltpu.VMEM((CHUNK, VW), jnp.float32),
        pltpu.VMEM_SHARED((NPAD, VW), jnp.float32),
        pltpu.SemaphoreType.DMA,
        pltpu.SemaphoreType.DMA,
    ]

    @functools.partial(pl.kernel, out_type=out_type, mesh=mesh,
                       scratch_types=scratch)
    def edge_kernel(kv_hbm, q_hbm, src_hbm, dst_hbm, et_hbm, zeros_hbm,
                    out_hbm, srcv, dstv, etv, gidxv, kvv, qdv, rowsv, acc_sh,
                    sem1, sem2):
        cid = lax.axis_index("c")
        sid = lax.axis_index("s")
        wid = sid * NC + cid
        pltpu.sync_copy(zeros_hbm.at[pl.ds(0, zrows)],
                        acc_sh.at[pl.ds(sid * zrows, zrows)])
        pltpu.sync_copy(zeros_hbm.at[pl.ds(0, CHUNK)], rowsv)
        plsc.subcore_barrier()

        iota = lax.iota(jnp.int32, LANES)
        perms = [iota ^ k for k in (8, 4, 2, 1)]
        onehots = [(1 - jnp.minimum(jnp.abs(iota - h), 1)).astype(jnp.float32)
                   for h in range(H)]

        def hsum(v):
            for p in perms:
                v = v + v.at[p].get(mode='promise_in_bounds')
            return v

        nchunks = base_ch + jnp.where(wid < extra_ch, 1, 0)

        def chunk_body(j, carry):
            base = (wid + j * NW) * CHUNK
            base = pl.multiple_of(base, 8)
            pltpu.sync_copy(src_hbm.at[pl.ds(base, CHUNK)], srcv)
            pltpu.sync_copy(dst_hbm.at[pl.ds(base, CHUNK)], dstv)
            pltpu.sync_copy(et_hbm.at[pl.ds(base, CHUNK)], etv)
            for i in range(CHUNK // LANES):
                sl = pl.ds(i * LANES, LANES)
                gidxv[sl] = etv[sl] * N + srcv[sl]
            cp1 = pltpu.async_copy(kv_hbm.at[gidxv], kvv, sem1)
            cp2 = pltpu.async_copy(q_hbm.at[dstv], qdv, sem2)
            cp1.wait()
            cp2.wait()

            def edge_body(e, c2):
                whs = []
                for h in range(H):
                    acc = None
                    for c in range(h * Dh // LANES, (h + 1) * Dh // LANES):
                        p = (kvv[e, pl.ds(c * LANES, LANES)] *
                             qdv[e, pl.ds(c * LANES, LANES)])
                        acc = p if acc is None else acc + p
                    whs.append(jnp.exp(hsum(acc)))
                for c in range(F // LANES):
                    vv = kvv[e, pl.ds(F + c * LANES, LANES)]
                    rowsv[e, pl.ds(c * LANES, LANES)] = (
                        vv * whs[(c * LANES) // Dh])
                wcol = None
                for h in range(H):
                    term = whs[h] * onehots[h]
                    wcol = term if wcol is None else wcol + term
                rowsv[e, pl.ds(F, LANES)] = wcol
                return c2

            lax.fori_loop(0, CHUNK, edge_body, 0)
            pltpu.sync_copy(rowsv, acc_sh.at[dstv], add=True)
            return carry

        lax.fori_loop(0, nchunks, chunk_body, 0)
        plsc.subcore_barrier()
        pltpu.sync_copy(acc_sh.at[pl.ds(sid * zrows, zrows)],
                        out_hbm.at[cid].at[pl.ds(sid * zrows, zrows)])

    return edge_kernel


def _t2_body(parts_ref, wn_ref, h0_ref, nt_ref, EXP_ref, Wa_ref, ba_ref,
             skip_ref, Wk_ref, bk_ref, Wq_ref, bq_ref, Wv_ref, bv_ref,
             K1_ref, V1_ref, q1_out, kv1_out):
    T = Wa_ref.shape[0]
    R = K1_ref.shape[0]
    F = EXP_ref.shape[1]
    p = parts_ref[0] + parts_ref[1]
    w = wn_ref[0] + wn_ref[1]
    den = jnp.dot(w, EXP_ref[...], preferred_element_type=jnp.float32) + 1e-9
    agg = p[:, :F] / den
    nt = nt_ref[...]
    masks = [(1 - jnp.minimum(jnp.abs(nt - t), 1)).astype(jnp.float32)
             for t in range(T)]

    def sel_proj(hin, W_ref, b_ref):
        out = None
        for t in range(T):
            m = jnp.dot(hin, W_ref[t], preferred_element_type=jnp.float32)
            m = (m + b_ref[t]) * masks[t]
            out = m if out is None else out + m
        return out

    out0 = sel_proj(agg, Wa_ref, ba_ref)
    sig = jax.nn.sigmoid(skip_ref[...])
    a = None
    for t in range(T):
        at = masks[t] * sig[t]
        a = at if a is None else a + at
    h1 = out0 * a + h0_ref[...] * (1.0 - a)

    k1 = sel_proj(h1, Wk_ref, bk_ref)
    q1 = sel_proj(h1, Wq_ref, bq_ref)
    v1 = sel_proj(h1, Wv_ref, bv_ref)
    B = q1.shape[0]
    F1 = K1_ref.shape[2]
    q1_out[...] = jnp.concatenate(
        [q1, jnp.zeros((B, VW - F1), jnp.float32)], axis=-1)
    zkv = jnp.zeros((B, VW - 2 * F1), jnp.float32)
    for r in range(R):
        kr = jnp.dot(k1, K1_ref[r], preferred_element_type=jnp.float32)
        vr = jnp.dot(v1, V1_ref[r], preferred_element_type=jnp.float32)
        kv1_out[r] = jnp.concatenate([kr, vr, zkv], axis=-1)


def _t2_call(parts, wn, h0, nt2, EXP, Wa, ba, skip2, Wk1, bk1, Wq1, bq1,
             Wv1, bv1, K1, V1):
    N = h0.shape[0]
    DIN = h0.shape[1]
    R = K1.shape[0]
    F1 = K1.shape[2]
    B = 1000 if N % 1000 == 0 else N
    grid = (N // B,)
    full = lambda a: pl.BlockSpec(a.shape, lambda i: (0,) * a.ndim)
    return pl.pallas_call(
        _t2_body,
        grid=grid,
        in_specs=[
            pl.BlockSpec((2, B, VW), lambda i: (0, i, 0)),
            pl.BlockSpec((2, B, LANES), lambda i: (0, i, 0)),
            pl.BlockSpec((B, DIN), lambda i: (i, 0)),
            pl.BlockSpec((B, 1), lambda i: (i, 0)),
            full(EXP), full(Wa), full(ba), full(skip2),
            full(Wk1), full(bk1), full(Wq1), full(bq1), full(Wv1), full(bv1),
            full(K1), full(V1),
        ],
        out_specs=[
            pl.BlockSpec((B, VW), lambda i: (i, 0)),
            pl.BlockSpec((R, B, VW), lambda i: (0, i, 0)),
        ],
        out_shape=[
            jax.ShapeDtypeStruct((N, VW), jnp.float32),
            jax.ShapeDtypeStruct((R, N, VW), jnp.float32),
        ],
    )(parts, wn, h0, nt2, EXP, Wa, ba, skip2, Wk1, bk1, Wq1, bq1, Wv1, bv1,
      K1, V1)


def _t3_body(parts_ref, nt_ref, SEL_ref, Wa_ref, ba_ref, out_ref):
    T = Wa_ref.shape[0]
    F = SEL_ref.shape[1]
    p = parts_ref[0] + parts_ref[1]
    den = jnp.dot(p, SEL_ref[...], preferred_element_type=jnp.float32) + 1e-9
    agg = p[:, :F] / den
    nt = nt_ref[...]
    out = None
    for t in range(T):
        mt = (1 - jnp.minimum(jnp.abs(nt - t), 1)).astype(jnp.float32)
        m = jnp.dot(agg, Wa_ref[t], preferred_element_type=jnp.float32)
        m = (m + ba_ref[t]) * mt
        out = m if out is None else out + m
    out_ref[...] = out


def _t3_call(parts, nt2, SEL, Wa, ba):
    N = nt2.shape[0]
    OUTF = Wa.shape[2]
    B = 1000 if N % 1000 == 0 else N
    grid = (N // B,)
    full = lambda a: pl.BlockSpec(a.shape, lambda i: (0,) * a.ndim)
    return pl.pallas_call(
        _t3_body,
        grid=grid,
        in_specs=[
            pl.BlockSpec((2, B, VW), lambda i: (0, i, 0)),
            pl.BlockSpec((B, 1), lambda i: (i, 0)),
            full(SEL), full(Wa), full(ba),
        ],
        out_specs=pl.BlockSpec((B, OUTF), lambda i: (i, 0)),
        out_shape=jax.ShapeDtypeStruct((N, OUTF), jnp.float32),
    )(parts, nt2, SEL, Wa, ba)


def _block_diag(W):
    R, H, D, _ = W.shape
    eye = jnp.eye(H, dtype=W.dtype)
    bd = W[:, :, :, None, :] * eye[None, :, None, :, None]
    return bd.reshape(R, H * D, H * D)


def _den_selector(F, H, rowW):
    Dh = F // H
    col_head = jnp.arange(F, dtype=jnp.int32) // Dh
    rows = jnp.arange(rowW, dtype=jnp.int32)
    sel = (rows[:, None] == (F + col_head[None, :])).astype(jnp.float32)
    return sel


def _den_expander(F, H):
    Dh = F // H
    col_head = jnp.arange(F, dtype=jnp.int32) // Dh
    rows = jnp.arange(LANES, dtype=jnp.int32)
    return (rows[:, None] == col_head[None, :]).astype(jnp.float32)


def kernel(x, node_type, edge_index, edge_type, lut_W, lut_b,
           Wk0, bk0, Wq0, bq0, Wv0, bv0, Watt0, Wmsg0, pri0, Wa0, ba0, skip0,
           Wk1, bk1, Wq1, bq1, Wv1, bv1, Watt1, Wmsg1, pri1, Wa1, ba1, skip1):
    N, DIN = x.shape
    E = edge_index.shape[1]
    R, H0_, D0_, _ = Watt0.shape
    _, H1_, D1_, _ = Watt1.shape
    F0 = H0_ * D0_
    F1 = H1_ * D1_

    nt2 = node_type.reshape(N, 1)
    src = edge_index[0]
    dst = edge_index[1]

    KBD0 = _block_diag(Watt0 * (pri0 / math.sqrt(D0_))[:, :, None, None])
    VBD0 = _block_diag(Wmsg0)
    K1 = _block_diag(Watt1 * (pri1 / math.sqrt(D1_))[:, :, None, None])
    V1 = _block_diag(Wmsg1)

    r3 = lambda b: b[:, None, :]
    q0, krot0, vrot0, h0 = _t1_call(
        x, nt2, lut_W, r3(lut_b), Wk0, r3(bk0), Wq0, r3(bq0), Wv0, r3(bv0),
        KBD0, VBD0)

    NPAD = _round_up(N, NS * 8 * 8)
    zeros = jnp.zeros((max(CHUNK, NPAD // NS), VW), jnp.float32)
    s1 = _make_sc_edge(N, NPAD, E, F0, H0_, CHUNK=64)
    sc1_out = s1(krot0.reshape(R * N, F0), vrot0.reshape(R * N, F0), q0,
                 src, dst, edge_type, zeros)
    if F0 + LANES > VW:
        parts1, wparts1 = sc1_out
        wn1 = wparts1.reshape(NC, NPAD, LANES)
    else:
        (parts1,) = sc1_out
        wn1 = parts1[:, :, F0:F0 + LANES]

    EXP0 = _den_expander(F0, H0_)
    q1p, kv1 = _t2_call(
        parts1[:, :N], wn1[:, :N], h0, nt2, EXP0, Wa0, r3(ba0),
        skip0[:, None], Wk1, r3(bk1), Wq1, r3(bq1), Wv1, r3(bv1), K1, V1)

    s2 = _make_sc_edge_packed(N, NPAD, E, F1, H1_, CHUNK=80)
    (parts2,) = s2(kv1.reshape(R * N, VW), q1p, src, dst, edge_type, zeros)

    SEL1 = _den_selector(F1, H1_, VW)
    out = _t3_call(parts2[:, :N], nt2, SEL1, Wa1, r3(ba1))
    return out
